# Initial kernel scaffold; baseline (speedup 1.0000x reference)
import jax
import jax.numpy as jnp
from jax.experimental import pallas as pl


def kernel(x, W1a, b1a, g1a, be1a, W1b, b1b, g1b, be1b, W1c, b1c, W2a, b2a, g2a, be2a, W2b, b2b):
    raise NotImplementedError("write your pallas kernel here")


# speedup vs baseline: 2.9608x; 2.9608x over previous
import functools

import jax
import jax.numpy as jnp
from jax.experimental import pallas as pl
from jax.experimental.pallas import tpu as pltpu

B, N, G, K, ED = 32, 2048, 128, 32, 384
RT = 1024
NROW = B * G * K
NT = NROW // RT
NG = B * G


def _fps_body(xt_ref, cen_ref, dists_ref):
    xx = xt_ref[0]
    yy = xt_ref[1]
    zz = xt_ref[2]
    cen_ref[...] = jnp.zeros((B, G, 8), jnp.float32)
    dists_ref[...] = jnp.full((B, N), 1e10, jnp.float32)
    iota = jax.lax.broadcasted_iota(jnp.int32, (B, N), 1)

    def body(i, far):
        m = iota == far
        cx = jnp.sum(jnp.where(m, xx, 0.0), axis=1, keepdims=True)
        cy = jnp.sum(jnp.where(m, yy, 0.0), axis=1, keepdims=True)
        cz = jnp.sum(jnp.where(m, zz, 0.0), axis=1, keepdims=True)
        cen_ref[:, pl.ds(i, 1), pl.ds(0, 1)] = cx[:, :, None]
        cen_ref[:, pl.ds(i, 1), pl.ds(1, 1)] = cy[:, :, None]
        cen_ref[:, pl.ds(i, 1), pl.ds(2, 1)] = cz[:, :, None]
        d = (xx - cx) ** 2 + (yy - cy) ** 2 + (zz - cz) ** 2
        nd = jnp.minimum(dists_ref[...], d)
        dists_ref[...] = nd
        mx = jnp.max(nd, axis=1, keepdims=True)
        far2 = jnp.min(jnp.where(nd == mx, iota, N), axis=1, keepdims=True)
        return far2

    jax.lax.fori_loop(0, G, body, jnp.zeros((B, 1), jnp.int32))


def _run_fps(xt):
    return pl.pallas_call(
        _fps_body,
        out_shape=jax.ShapeDtypeStruct((B, G, 8), jnp.float32),
        scratch_shapes=[pltpu.VMEM((B, N), jnp.float32)],
    )(xt)


def _knn_body(xt_ref, cen_ref, oidx_ref, d_ref):
    b = pl.program_id(0)
    x0 = xt_ref[0, pl.ds(0, 1), :]
    x1 = xt_ref[0, pl.ds(1, 1), :]
    x2 = xt_ref[0, pl.ds(2, 1), :]
    cen = cen_ref[0]
    c0 = cen[:, 0:1]
    c1 = cen[:, 1:2]
    c2 = cen[:, 2:3]
    d_ref[...] = (c0 - x0) ** 2 + (c1 - x1) ** 2 + (c2 - x2) ** 2
    iota = jax.lax.broadcasted_iota(jnp.int32, (G, N), 1)
    base = b * N
    for k in range(K):
        dd = d_ref[...]
        mn = jnp.min(dd, axis=1, keepdims=True)
        idx = jnp.min(jnp.where(dd == mn, iota, N), axis=1, keepdims=True)
        oidx_ref[0, :, pl.ds(k, 1)] = idx + base
        d_ref[...] = jnp.where(iota == idx, jnp.float32(1e30), dd)


def _run_knn(xbt, cen8):
    return pl.pallas_call(
        _knn_body,
        grid=(B,),
        in_specs=[
            pl.BlockSpec((1, 3, N), lambda b: (b, 0, 0)),
            pl.BlockSpec((1, G, 8), lambda b: (b, 0, 0)),
        ],
        out_specs=pl.BlockSpec((1, G, K), lambda b: (b, 0, 0)),
        out_shape=jax.ShapeDtypeStruct((B, G, K), jnp.int32),
        scratch_shapes=[pltpu.VMEM((G, N), jnp.float32)],
    )(xbt, cen8)


def _morton_body(ccol_ref, crow_ref, sidx_ref):
    b = pl.program_id(0)
    ccol = ccol_ref[0]
    crow = crow_ref[0]
    icol = jax.lax.broadcasted_iota(jnp.int32, (G, G), 0)
    jrow = jax.lax.broadcasted_iota(jnp.int32, (G, G), 1)
    cmp = (crow < ccol) | ((crow == ccol) & (jrow < icol))
    rank = jnp.sum(cmp.astype(jnp.int32), axis=1, keepdims=True)
    eq = (rank == jrow).astype(jnp.int32)
    ordv = jnp.sum(eq * icol, axis=0, keepdims=True)
    sidx_ref[0] = ordv + b * G


def _run_morton(cen8):
    center = cen8[:, :, :3]
    mn = center.min(axis=1, keepdims=True)
    mx = center.max(axis=1, keepdims=True)
    q = jnp.clip((center - mn) / (mx - mn + 1e-9) * 1023.0, 0, 1023)
    q = q.astype(jnp.int32)
    code = jnp.zeros((B, G), jnp.int32)
    for bit in range(10):
        for a in range(3):
            code = code | (((q[..., a] >> bit) & 1) << (3 * bit + a))
    return pl.pallas_call(
        _morton_body,
        grid=(B,),
        in_specs=[
            pl.BlockSpec((1, G, 1), lambda b: (b, 0, 0)),
            pl.BlockSpec((1, 1, G), lambda b: (b, 0, 0)),
        ],
        out_specs=pl.BlockSpec((1, 1, G), lambda b: (b, 0, 0)),
        out_shape=jax.ShapeDtypeStruct((B, 1, G), jnp.int32),
    )(code[:, :, None], code[:, None, :])


def _dot(a, b):
    return jax.lax.dot_general(
        a, b, (((1,), (1,)), ((), ())), preferred_element_type=jnp.float32)


def _sa_body(nb_ref, cc_ref, w1_ref, b1_ref, s_ref, ss_ref):
    @pl.when(pl.program_id(0) == 0)
    def _():
        s_ref[...] = jnp.zeros((1, 256), jnp.float32)
        ss_ref[...] = jnp.zeros((1, 256), jnp.float32)

    pg = nb_ref[...] + cc_ref[...]
    z1 = _dot(pg, w1_ref[...]) + b1_ref[...]
    s_ref[...] += jnp.sum(z1, axis=0, keepdims=True)
    ss_ref[...] += jnp.sum(z1 * z1, axis=0, keepdims=True)


def _run_sa(nb8, ccfull, w1, b1):
    return pl.pallas_call(
        _sa_body,
        grid=(NT,),
        in_specs=[
            pl.BlockSpec((RT, 8), lambda i: (i, 0)),
            pl.BlockSpec((RT, 8), lambda i: (i, 0)),
            pl.BlockSpec((256, 8), lambda i: (0, 0)),
            pl.BlockSpec((1, 256), lambda i: (0, 0)),
        ],
        out_specs=[
            pl.BlockSpec((1, 256), lambda i: (0, 0)),
            pl.BlockSpec((1, 256), lambda i: (0, 0)),
        ],
        out_shape=[
            jax.ShapeDtypeStruct((1, 256), jnp.float32),
            jax.ShapeDtypeStruct((1, 256), jnp.float32),
        ],
    )(nb8, ccfull, w1, b1)


def _sb_body(nb_ref, cc_ref, w1_ref, b1_ref, s1_ref, t1_ref, w2_ref, b2_ref,
             s_ref, ss_ref):
    @pl.when(pl.program_id(0) == 0)
    def _():
        s_ref[...] = jnp.zeros((1, 512), jnp.float32)
        ss_ref[...] = jnp.zeros((1, 512), jnp.float32)

    pg = nb_ref[...] + cc_ref[...]
    z1 = _dot(pg, w1_ref[...]) + b1_ref[...]
    h1 = jnp.maximum(z1 * s1_ref[...] + t1_ref[...], 0.0)
    z2 = _dot(h1, w2_ref[...]) + b2_ref[...]
    s_ref[...] += jnp.sum(z2, axis=0, keepdims=True)
    ss_ref[...] += jnp.sum(z2 * z2, axis=0, keepdims=True)


def _run_sb(nb8, ccfull, w1, b1, s1, t1, w2, b2):
    return pl.pallas_call(
        _sb_body,
        grid=(NT,),
        in_specs=[
            pl.BlockSpec((RT, 8), lambda i: (i, 0)),
            pl.BlockSpec((RT, 8), lambda i: (i, 0)),
            pl.BlockSpec((256, 8), lambda i: (0, 0)),
            pl.BlockSpec((1, 256), lambda i: (0, 0)),
            pl.BlockSpec((1, 256), lambda i: (0, 0)),
            pl.BlockSpec((1, 256), lambda i: (0, 0)),
            pl.BlockSpec((512, 256), lambda i: (0, 0)),
            pl.BlockSpec((1, 512), lambda i: (0, 0)),
        ],
        out_specs=[
            pl.BlockSpec((1, 512), lambda i: (0, 0)),
            pl.BlockSpec((1, 512), lambda i: (0, 0)),
        ],
        out_shape=[
            jax.ShapeDtypeStruct((1, 512), jnp.float32),
            jax.ShapeDtypeStruct((1, 512), jnp.float32),
        ],
    )(nb8, ccfull, w1, b1, s1, t1, w2, b2)


def _mid_body(nb_ref, cc_ref, w1_ref, b1_ref, s1_ref, t1_ref, w2_ref, b2_ref,
              s2_ref, t2_ref, w3_ref, b3_ref, wf_ref, wh_ref, b4_ref,
              h3_ref, fg_ref, s_ref, ss_ref):
    @pl.when(pl.program_id(0) == 0)
    def _():
        s_ref[...] = jnp.zeros((1, 2 * ED), jnp.float32)
        ss_ref[...] = jnp.zeros((1, 2 * ED), jnp.float32)

    pg = nb_ref[...] + cc_ref[...]
    z1 = _dot(pg, w1_ref[...]) + b1_ref[...]
    h1 = jnp.maximum(z1 * s1_ref[...] + t1_ref[...], 0.0)
    z2 = _dot(h1, w2_ref[...]) + b2_ref[...]
    h2 = jnp.maximum(z2 * s2_ref[...] + t2_ref[...], 0.0)
    h3 = _dot(h2, w3_ref[...]) + b3_ref[...]
    h3_ref[...] = h3
    h3g = h3.reshape(RT // K, K, ED)
    fg = jnp.max(h3g, axis=1)
    fg_ref[...] = fg
    zf = _dot(fg, wf_ref[...])
    zfe = jnp.broadcast_to(zf[:, None, :], (RT // K, K, 2 * ED))
    z4 = zfe.reshape(RT, 2 * ED) + _dot(h3, wh_ref[...]) + b4_ref[...]
    s_ref[...] += jnp.sum(z4, axis=0, keepdims=True)
    ss_ref[...] += jnp.sum(z4 * z4, axis=0, keepdims=True)


def _run_mid(nb8, ccfull, w1, b1, s1, t1, w2, b2, s2, t2, w3, b3, wf, wh, b4):
    return pl.pallas_call(
        _mid_body,
        grid=(NT,),
        in_specs=[
            pl.BlockSpec((RT, 8), lambda i: (i, 0)),
            pl.BlockSpec((RT, 8), lambda i: (i, 0)),
            pl.BlockSpec((256, 8), lambda i: (0, 0)),
            pl.BlockSpec((1, 256), lambda i: (0, 0)),
            pl.BlockSpec((1, 256), lambda i: (0, 0)),
            pl.BlockSpec((1, 256), lambda i: (0, 0)),
            pl.BlockSpec((512, 256), lambda i: (0, 0)),
            pl.BlockSpec((1, 512), lambda i: (0, 0)),
            pl.BlockSpec((1, 512), lambda i: (0, 0)),
            pl.BlockSpec((1, 512), lambda i: (0, 0)),
            pl.BlockSpec((ED, 512), lambda i: (0, 0)),
            pl.BlockSpec((1, ED), lambda i: (0, 0)),
            pl.BlockSpec((2 * ED, ED), lambda i: (0, 0)),
            pl.BlockSpec((2 * ED, ED), lambda i: (0, 0)),
            pl.BlockSpec((1, 2 * ED), lambda i: (0, 0)),
        ],
        out_specs=[
            pl.BlockSpec((RT, ED), lambda i: (i, 0)),
            pl.BlockSpec((RT // K, ED), lambda i: (i, 0)),
            pl.BlockSpec((1, 2 * ED), lambda i: (0, 0)),
            pl.BlockSpec((1, 2 * ED), lambda i: (0, 0)),
        ],
        out_shape=[
            jax.ShapeDtypeStruct((NROW, ED), jnp.float32),
            jax.ShapeDtypeStruct((NG, ED), jnp.float32),
            jax.ShapeDtypeStruct((1, 2 * ED), jnp.float32),
            jax.ShapeDtypeStruct((1, 2 * ED), jnp.float32),
        ],
    )(nb8, ccfull, w1, b1, s1, t1, w2, b2, s2, t2, w3, b3, wf, wh, b4)


def _fin_body(h3_ref, fg_ref, wf_ref, wh_ref, b4_ref, s4_ref, t4_ref,
              w5_ref, b5_ref, o_ref):
    fg = fg_ref[...]
    zf = _dot(fg, wf_ref[...])
    zfe = jnp.broadcast_to(zf[:, None, :], (RT // K, K, 2 * ED))
    z4 = zfe.reshape(RT, 2 * ED) + _dot(h3_ref[...], wh_ref[...]) + b4_ref[...]
    h4 = jnp.maximum(z4 * s4_ref[...] + t4_ref[...], 0.0)
    z5 = _dot(h4, w5_ref[...]) + b5_ref[...]
    o_ref[...] = jnp.max(z5.reshape(RT // K, K, ED), axis=1)


def _run_fin(h3, fg, wf, wh, b4, s4, t4, w5, b5):
    return pl.pallas_call(
        _fin_body,
        grid=(NT,),
        in_specs=[
            pl.BlockSpec((RT, ED), lambda i: (i, 0)),
            pl.BlockSpec((RT // K, ED), lambda i: (i, 0)),
            pl.BlockSpec((2 * ED, ED), lambda i: (0, 0)),
            pl.BlockSpec((2 * ED, ED), lambda i: (0, 0)),
            pl.BlockSpec((1, 2 * ED), lambda i: (0, 0)),
            pl.BlockSpec((1, 2 * ED), lambda i: (0, 0)),
            pl.BlockSpec((1, 2 * ED), lambda i: (0, 0)),
            pl.BlockSpec((ED, 2 * ED), lambda i: (0, 0)),
            pl.BlockSpec((1, ED), lambda i: (0, 0)),
        ],
        out_specs=pl.BlockSpec((RT // K, ED), lambda i: (i, 0)),
        out_shape=jax.ShapeDtypeStruct((NG, ED), jnp.float32),
    )(h3, fg, wf, wh, b4, s4, t4, w5, b5)


def kernel(x, W1a, b1a, g1a, be1a, W1b, b1b, g1b, be1b, W1c, b1c,
           W2a, b2a, g2a, be2a, W2b, b2b):
    nrow = jnp.float32(NROW)
    xt = jnp.transpose(x, (2, 0, 1))
    xbt = jnp.transpose(x, (0, 2, 1))
    cen8 = _run_fps(xt)
    flatidx = _run_knn(xbt, cen8)
    sidx = _run_morton(cen8)

    x8 = jnp.pad(x.reshape(B * N, 3), ((0, 0), (0, 5)))
    nb8 = x8[flatidx.reshape(-1)]

    cenf = cen8.reshape(NG, 8)[:, :3]
    cencomb = jnp.concatenate(
        [-cenf, cenf, jnp.zeros((NG, 2), jnp.float32)], axis=1)
    ccfull = jnp.repeat(cencomb, K, axis=0)

    w1a8 = jnp.concatenate(
        [W1a[:, :3], W1a[:, 3:], jnp.zeros((256, 2), jnp.float32)], axis=1)

    def fold(s, ss, g, be):
        m = s[0] / nrow
        v = ss[0] / nrow - m * m
        sc = g / jnp.sqrt(v + 1e-5)
        return sc[None, :], (be - m * sc)[None, :]

    sum1, sumsq1 = _run_sa(nb8, ccfull, w1a8, b1a[None, :])
    s1, t1 = fold(sum1, sumsq1, g1a, be1a)

    sum2, sumsq2 = _run_sb(nb8, ccfull, w1a8, b1a[None, :], s1, t1,
                           W1b, b1b[None, :])
    s2, t2 = fold(sum2, sumsq2, g1b, be1b)

    w2af = W2a[:, :ED]
    w2ah = W2a[:, ED:]
    h3, fg, sum4, sumsq4 = _run_mid(
        nb8, ccfull, w1a8, b1a[None, :], s1, t1, W1b, b1b[None, :], s2, t2,
        W1c, b1c[None, :], w2af, w2ah, b2a[None, :])
    s4, t4 = fold(sum4, sumsq4, g2a, be2a)

    mlp_out = _run_fin(h3, fg, w2af, w2ah, b2a[None, :], s4, t4,
                       W2b, b2b[None, :])

    out = mlp_out[sidx.reshape(-1)]
    return out.reshape(B, G, ED)
